# baseline (speedup 1.0000x reference)
import jax
import jax.numpy as jnp
from jax import lax
from jax.experimental import pallas as pl

N = 2048
D_IN = 256
D = 256
D_HEAD = 64
H = 4
HALF = 64
KTOP = 32
TILE = 256
SLAB = 512
NT = N // TILE
SEG = 1024
INV_TEMP_SCALE = 0.25


def _body(x_ref, wq_ref, bq_ref, wk_ref, bk_ref, out_ref):
    t = pl.program_id(0)
    i0 = t * TILE
    seg_start = (t // (SEG // TILE)) * SEG
    s0 = pl.multiple_of(
        jnp.clip(TILE * t - 2 * HALF, seg_start, seg_start + SEG - SLAB), 128)

    dn = (((1,), (1,)), ((), ()))
    xq = x_ref[pl.ds(i0, TILE), :]
    q = lax.dot_general(xq, wq_ref[...], dn,
                        preferred_element_type=jnp.float32) + bq_ref[0, :][None, :]
    xk = x_ref[pl.ds(s0, SLAB), :]
    k = lax.dot_general(xk, wk_ref[...], dn,
                        preferred_element_type=jnp.float32) + bk_ref[0, :][None, :]

    cols = s0 + lax.broadcasted_iota(jnp.int32, (SLAB, TILE), 0)
    rows = i0 + lax.broadcasted_iota(jnp.int32, (SLAB, TILE), 1)
    valid = (cols >= rows - HALF) & (cols < rows + HALF)

    attn = jnp.zeros((SLAB, TILE), jnp.float32)
    for h in range(H):
        qh = q[:, h * D_HEAD:(h + 1) * D_HEAD]
        kh = k[:, h * D_HEAD:(h + 1) * D_HEAD]
        s = lax.dot_general(kh, qh, dn, preferred_element_type=jnp.float32,
                            precision=lax.Precision.HIGHEST)
        z = jnp.where(valid, s * INV_TEMP_SCALE, -1e30)
        m = jnp.max(z, axis=0, keepdims=True)
        p = jnp.exp(z - m)
        attn = attn + p / jnp.sum(p, axis=0, keepdims=True)
    attn = jnp.where(valid, attn * (1.0 / H), 0.0)

    u = lax.bitcast_convert_type(attn, jnp.int32)
    lo = jnp.zeros((1, TILE), jnp.int32)
    hi = jnp.full((1, TILE), 0x3D800000, jnp.int32)
    def _mid(l, h):
        return l + ((h - l + 1) >> 1)

    def _ge(thr):
        return jnp.sum((u >= thr).astype(jnp.int32), axis=0,
                       keepdims=True) >= KTOP

    for _ in range(15):
        t2 = _mid(lo, hi)
        t1 = _mid(lo, t2 - 1)
        t3 = _mid(t2, hi)
        c1, c2, c3 = _ge(t1), _ge(t2), _ge(t3)
        lo = jnp.where(c2, jnp.where(c3, t3, t2), jnp.where(c1, t1, lo))
        hi = jnp.where(c2, jnp.where(c3, hi, t3 - 1),
                       jnp.where(c1, t2 - 1, t1 - 1))
    tau = lo
    gt = u > tau
    eq = u == tau
    cg = jnp.sum(gt.astype(jnp.float32), axis=0, keepdims=True)
    ii = lax.broadcasted_iota(jnp.int32, (SLAB, SLAB), 0)
    jj = lax.broadcasted_iota(jnp.int32, (SLAB, SLAB), 1)
    tri = (ii >= jj).astype(jnp.float32)
    rank_eq = lax.dot_general(tri, eq.astype(jnp.float32),
                              (((1,), (0,)), ((), ())),
                              preferred_element_type=jnp.float32)
    sel = gt | (eq & (rank_eq <= (KTOP - cg)))

    out_ref[...] = jnp.zeros((TILE, N), jnp.float32)
    out_ref[:, pl.ds(s0, SLAB)] = sel.astype(jnp.float32).T


def kernel(node_features, Wq, bq, Wk, bk, Wv, bv, graph_num_nodes,
           num_relation):
    del Wv, bv, graph_num_nodes, num_relation
    bq2 = bq.reshape(1, D)
    bk2 = bk.reshape(1, D)
    return pl.pallas_call(
        _body,
        grid=(NT,),
        in_specs=[
            pl.BlockSpec((N, D_IN), lambda t: (0, 0)),
            pl.BlockSpec((D, D_IN), lambda t: (0, 0)),
            pl.BlockSpec((1, D), lambda t: (0, 0)),
            pl.BlockSpec((D, D_IN), lambda t: (0, 0)),
            pl.BlockSpec((1, D), lambda t: (0, 0)),
        ],
        out_specs=pl.BlockSpec((TILE, N), lambda t: (t, 0)),
        out_shape=jax.ShapeDtypeStruct((N, N), jnp.float32),
    )(node_features, Wq, bq2, Wk, bk2)


# speedup vs baseline: 1.0910x; 1.0910x over previous
import jax
import jax.numpy as jnp
from jax import lax
from jax.experimental import pallas as pl

N = 2048
D_IN = 256
D = 256
D_HEAD = 64
H = 4
HALF = 64
KTOP = 32
TILE = 256
SLAB = 512
NT = N // TILE
SEG = 1024
INV_TEMP_SCALE = 0.25


def _body(x_ref, wq_ref, bq_ref, wk_ref, bk_ref, out_ref):
    t = pl.program_id(0)
    i0 = t * TILE
    seg_start = (t // (SEG // TILE)) * SEG
    s0 = pl.multiple_of(
        jnp.clip(TILE * t - 2 * HALF, seg_start, seg_start + SEG - SLAB), 128)

    dn = (((1,), (1,)), ((), ()))
    xq = x_ref[pl.ds(i0, TILE), :]
    q = lax.dot_general(xq, wq_ref[...], dn,
                        preferred_element_type=jnp.float32) + bq_ref[0, :][None, :]
    xk = x_ref[pl.ds(s0, SLAB), :]
    k = lax.dot_general(xk, wk_ref[...], dn,
                        preferred_element_type=jnp.float32) + bk_ref[0, :][None, :]

    cols = s0 + lax.broadcasted_iota(jnp.int32, (SLAB, TILE), 0)
    rows = i0 + lax.broadcasted_iota(jnp.int32, (SLAB, TILE), 1)
    valid = (cols >= rows - HALF) & (cols < rows + HALF)

    attn = jnp.zeros((SLAB, TILE), jnp.float32)
    for h in range(H):
        qh = q[:, h * D_HEAD:(h + 1) * D_HEAD]
        kh = k[:, h * D_HEAD:(h + 1) * D_HEAD]
        s = lax.dot_general(kh, qh, dn, preferred_element_type=jnp.float32,
                            precision=lax.Precision.HIGHEST)
        z = jnp.where(valid, s * INV_TEMP_SCALE, -1e30)
        m = jnp.max(z, axis=0, keepdims=True)
        p = jnp.exp(z - m)
        attn = attn + p / jnp.sum(p, axis=0, keepdims=True)
    attn = jnp.where(valid, attn * (1.0 / H), 0.0)

    u = lax.bitcast_convert_type(attn, jnp.int32)
    lo = jnp.zeros((1, TILE), jnp.int32)
    hi = jnp.full((1, TILE), 0x3D800000, jnp.int32)
    for _ in range(30):
        mid = lo + ((hi - lo + 1) >> 1)
        c = jnp.sum((u >= mid).astype(jnp.int32), axis=0, keepdims=True)
        ge = c >= KTOP
        lo = jnp.where(ge, mid, lo)
        hi = jnp.where(ge, hi, mid - 1)
    tau = lo
    gt = u > tau
    eq = u == tau
    cg = jnp.sum(gt.astype(jnp.float32), axis=0, keepdims=True)
    ii = lax.broadcasted_iota(jnp.int32, (SLAB, SLAB), 0)
    jj = lax.broadcasted_iota(jnp.int32, (SLAB, SLAB), 1)
    tri = (ii >= jj).astype(jnp.float32)
    rank_eq = lax.dot_general(tri, eq.astype(jnp.float32),
                              (((1,), (0,)), ((), ())),
                              preferred_element_type=jnp.float32)
    sel = gt | (eq & (rank_eq <= (KTOP - cg)))

    out_ref[...] = jnp.zeros((TILE, N), jnp.float32)
    out_ref[:, pl.ds(s0, SLAB)] = sel.astype(jnp.float32).T


def kernel(node_features, Wq, bq, Wk, bk, Wv, bv, graph_num_nodes,
           num_relation):
    del Wv, bv, graph_num_nodes, num_relation
    bq2 = bq.reshape(1, D)
    bk2 = bk.reshape(1, D)
    return pl.pallas_call(
        _body,
        grid=(NT,),
        in_specs=[
            pl.BlockSpec((N, D_IN), lambda t: (0, 0)),
            pl.BlockSpec((D, D_IN), lambda t: (0, 0)),
            pl.BlockSpec((1, D), lambda t: (0, 0)),
            pl.BlockSpec((D, D_IN), lambda t: (0, 0)),
            pl.BlockSpec((1, D), lambda t: (0, 0)),
        ],
        out_specs=pl.BlockSpec((TILE, N), lambda t: (t, 0)),
        out_shape=jax.ShapeDtypeStruct((N, N), jnp.float32),
    )(node_features, Wq, bq2, Wk, bk2)
